# baseline (speedup 1.0000x reference)
import functools

import jax
import jax.numpy as jnp
from jax import lax
from jax.experimental import pallas as pl
from jax.experimental.pallas import tpu as pltpu
from jax.experimental.pallas import tpu_sc as plsc

N = 10000
E = 320000
D = 128
G = 64
NC = 2
NS = 16
NW = NC * NS
EPW = E // NW
CH = 100
NCHUNK = EPW // CH
DW = 16
RPT = 624
RTAIL = N - NS * RPT
RB = 2000
NRB = N // RB

def _dot(a, b):
    return jnp.dot(a, b, preferred_element_type=jnp.float32)


@functools.lru_cache(maxsize=None)
def _make_prop():
    mesh = plsc.VectorSubcoreMesh(core_axis_name="c", subcore_axis_name="s")

    @functools.partial(
        pl.kernel,
        out_type=jax.ShapeDtypeStruct((NC * N, D), jnp.float32),
        mesh=mesh,
        scratch_types=[
            pltpu.VMEM_SHARED((N, D), jnp.float32),
            [pltpu.VMEM((CH,), jnp.int32) for _ in range(2)],
            [pltpu.VMEM((CH,), jnp.int32) for _ in range(2)],
            [pltpu.VMEM((CH, D), jnp.float32) for _ in range(2)],
            [pltpu.SemaphoreType.DMA for _ in range(2)],
            [pltpu.SemaphoreType.DMA for _ in range(2)],
        ],
    )
    def prop(u_hbm, src3, dst3, zeros_hbm, out_hbm,
             acc, sidx, didx, rows, gsem, isem):
        cid = lax.axis_index("c")
        sid = lax.axis_index("s")
        wid = sid * NC + cid
        row0 = sid * RPT

        def stage_idx(j, q):
            pltpu.async_copy(src3.at[wid, j], sidx[q], isem[q])
            pltpu.async_copy(dst3.at[wid, j], didx[q], isem[q])

        def wait_idx(j, q):
            pltpu.make_async_copy(src3.at[wid, j], sidx[q], isem[q]).wait()
            pltpu.make_async_copy(dst3.at[wid, j], didx[q], isem[q]).wait()

        stage_idx(0, 0)
        stage_idx(1, 1)
        pltpu.sync_copy(zeros_hbm.at[pl.ds(row0, RPT)],
                        acc.at[pl.ds(row0, RPT)])

        @pl.when(sid == 0)
        def _():
            pltpu.sync_copy(zeros_hbm.at[pl.ds(NS * RPT, RTAIL)],
                            acc.at[pl.ds(NS * RPT, RTAIL)])

        plsc.subcore_barrier()
        wait_idx(0, 0)
        pltpu.async_copy(u_hbm.at[sidx[0]], rows[0], gsem[0])

        def step(k, carry):
            j0 = 2 * k
            j1 = 2 * k + 1

            @pl.when(j1 < NCHUNK)
            def _():
                wait_idx(j1, 1)
                pltpu.async_copy(u_hbm.at[sidx[1]], rows[1], gsem[1])

            pltpu.make_async_copy(u_hbm.at[sidx[0]], rows[0], gsem[0]).wait()
            pltpu.sync_copy(rows[0], acc.at[didx[0]], add=True)

            @pl.when(j0 + 2 < NCHUNK)
            def _():
                stage_idx(j0 + 2, 0)

            @pl.when(j1 < NCHUNK)
            def _():
                pltpu.make_async_copy(u_hbm.at[sidx[1]], rows[1],
                                      gsem[1]).wait()
                pltpu.sync_copy(rows[1], acc.at[didx[1]], add=True)

            @pl.when(j0 + 2 < NCHUNK)
            def _():
                wait_idx(j0 + 2, 0)
                pltpu.async_copy(u_hbm.at[sidx[0]], rows[0], gsem[0])

            @pl.when(j1 + 2 < NCHUNK)
            def _():
                stage_idx(j1 + 2, 1)

            return carry

        lax.fori_loop(0, (NCHUNK + 1) // 2, step, 0)
        plsc.subcore_barrier()
        pltpu.sync_copy(acc.at[pl.ds(row0, RPT)],
                        out_hbm.at[pl.ds(cid * N + row0, RPT)])

        @pl.when(sid == 0)
        def _():
            pltpu.sync_copy(acc.at[pl.ds(NS * RPT, RTAIL)],
                            out_hbm.at[pl.ds(cid * N + NS * RPT, RTAIL)])

    return prop


def _prelude_body(d0, d1, x, dinv_ref, u0_ref):
    deg = d0[...] + d1[...]
    dinv = jnp.where(deg > 0.0, lax.rsqrt(deg), 0.0)
    dinv_ref[...] = dinv
    u0_ref[...] = dinv * x[...]


def _tc_prelude(Dp, x):
    return pl.pallas_call(
        _prelude_body,
        grid=(NRB,),
        in_specs=[
            pl.BlockSpec((RB, D), lambda r: (r, 0)),
            pl.BlockSpec((RB, D), lambda r: (r + NRB, 0)),
            pl.BlockSpec((RB, D), lambda r: (r, 0)),
        ],
        out_specs=[
            pl.BlockSpec((RB, D), lambda r: (r, 0)),
            pl.BlockSpec((RB, D), lambda r: (r, 0)),
        ],
        out_shape=[
            jax.ShapeDtypeStruct((N, D), jnp.float32),
            jax.ShapeDtypeStruct((N, D), jnp.float32),
        ],
    )(Dp, Dp, x)


def _mid_body(p0, p1, dv, h, w0, w1, u1_ref, acc_ref):
    tx1 = -dv[...] * (p0[...] + p1[...])
    u1_ref[...] = dv[...] * tx1
    acc_ref[...] = _dot(h[...], w0[...]) + _dot(tx1, w1[...])


def _tc_mid(P, dinvb, h, W0, W1):
    return pl.pallas_call(
        _mid_body,
        grid=(NRB,),
        in_specs=[
            pl.BlockSpec((RB, D), lambda r: (r, 0)),
            pl.BlockSpec((RB, D), lambda r: (r + NRB, 0)),
            pl.BlockSpec((RB, D), lambda r: (r, 0)),
            pl.BlockSpec((RB, D), lambda r: (r, 0)),
            pl.BlockSpec((D, D), lambda r: (0, 0)),
            pl.BlockSpec((D, D), lambda r: (0, 0)),
        ],
        out_specs=[
            pl.BlockSpec((RB, D), lambda r: (r, 0)),
            pl.BlockSpec((RB, D), lambda r: (r, 0)),
        ],
        out_shape=[
            jax.ShapeDtypeStruct((N, D), jnp.float32),
            jax.ShapeDtypeStruct((N, D), jnp.float32),
        ],
    )(P, P, dinvb, h, W0, W1)


def _end_body(q0, q1, dv, h, acc, w2, b, alpha, hn_ref, un_ref):
    tx2 = -2.0 * dv[...] * (q0[...] + q1[...]) - h[...]
    out = acc[...] + _dot(tx2, w2[...]) + b[...]
    a = alpha[0, 0]
    hn = jnp.where(out >= 0.0, out, a * out)
    hn_ref[...] = hn
    un_ref[...] = dv[...] * hn


def _tc_end(Q, dinvb, h, acc0, W2, b, alpha):
    return pl.pallas_call(
        _end_body,
        grid=(NRB,),
        in_specs=[
            pl.BlockSpec((RB, D), lambda r: (r, 0)),
            pl.BlockSpec((RB, D), lambda r: (r + NRB, 0)),
            pl.BlockSpec((RB, D), lambda r: (r, 0)),
            pl.BlockSpec((RB, D), lambda r: (r, 0)),
            pl.BlockSpec((RB, D), lambda r: (r, 0)),
            pl.BlockSpec((D, D), lambda r: (0, 0)),
            pl.BlockSpec((1, D), lambda r: (0, 0)),
            pl.BlockSpec(memory_space=pltpu.SMEM),
        ],
        out_specs=[
            pl.BlockSpec((RB, D), lambda r: (r, 0)),
            pl.BlockSpec((RB, D), lambda r: (r, 0)),
        ],
        out_shape=[
            jax.ShapeDtypeStruct((N, D), jnp.float32),
            jax.ShapeDtypeStruct((N, D), jnp.float32),
        ],
    )(Q, Q, dinvb, h, acc0, W2, b, alpha)


def _head_body(h, bat, w1, b1, w2, b2, w3, b3, a3, out_ref, g_ref):
    r = pl.program_id(0)

    @pl.when(r == 0)
    def _():
        g_ref[...] = jnp.zeros_like(g_ref)

    bblk = bat[0, 0, :]
    onehot = (lax.broadcasted_iota(jnp.int32, (G, RB), 0)
              == bblk[None, :]).astype(jnp.float32)
    g_ref[...] += _dot(onehot, h[...])

    @pl.when(r == NRB - 1)
    def _():
        g = g_ref[...]
        z = _dot(g, w1[...]) + b1[...]
        a = a3[0, 0]
        z = jnp.where(z >= 0.0, z, a * z)
        z = _dot(z, w2[...]) + b2[...]
        z = 1.0 / (1.0 + jnp.exp(-z))
        z = _dot(z, w3[...]) + b3[...]
        m = jnp.max(z, axis=-1, keepdims=True)
        e = jnp.exp(z - m)
        out_ref[...] = (z - m) - jnp.log(jnp.sum(e, axis=-1, keepdims=True))


def _tc_head(h, batch3, W1, b1, W2, b2, W3, b3, a3):
    return pl.pallas_call(
        _head_body,
        grid=(NRB,),
        in_specs=[
            pl.BlockSpec((RB, D), lambda r: (r, 0)),
            pl.BlockSpec((1, 1, RB), lambda r: (r, 0, 0)),
            pl.BlockSpec((D, D), lambda r: (0, 0)),
            pl.BlockSpec((1, D), lambda r: (0, 0)),
            pl.BlockSpec((D, D // 2), lambda r: (0, 0)),
            pl.BlockSpec((1, D // 2), lambda r: (0, 0)),
            pl.BlockSpec((D // 2, 10), lambda r: (0, 0)),
            pl.BlockSpec((1, 10), lambda r: (0, 0)),
            pl.BlockSpec(memory_space=pltpu.SMEM),
        ],
        out_specs=pl.BlockSpec((G, 10), lambda r: (0, 0)),
        out_shape=jax.ShapeDtypeStruct((G, 10), jnp.float32),
        scratch_shapes=[pltpu.VMEM((G, D), jnp.float32)],
    )(h, batch3, W1, b1, W2, b2, W3, b3, a3)


def kernel(x, edge_index, batch, Wc, bc, a1, a3,
           W_fc1, b_fc1, W_fc2, b_fc2, W_fc3, b_fc3):
    src3 = edge_index[0].astype(jnp.int32).reshape(NW, NCHUNK, CH)
    dst3 = edge_index[1].astype(jnp.int32).reshape(NW, NCHUNK, CH)
    zeros = jnp.zeros((N, D), jnp.float32)
    ones = jnp.ones((N, D), jnp.float32)
    prop = _make_prop()

    Dp = prop(ones, src3, src3, zeros)
    dinvb, u = _tc_prelude(Dp, x)

    h = x
    for i in range(4):
        P = prop(u, src3, dst3, zeros)
        alpha = a1 if i == 0 else jnp.float32(0.0)
        u1, acc0 = _tc_mid(P, dinvb, h, Wc[i, 0], Wc[i, 1])
        Q = prop(u1, src3, dst3, zeros)
        h, u = _tc_end(Q, dinvb, h, acc0, Wc[i, 2],
                       bc[i].reshape(1, D), jnp.reshape(alpha, (1, 1)))

    batch3 = batch.reshape(NRB, 1, RB).astype(jnp.int32)
    return _tc_head(h, batch3, W_fc1, b_fc1.reshape(1, D),
                    W_fc2, b_fc2.reshape(1, D // 2),
                    W_fc3, b_fc3.reshape(1, 10), jnp.reshape(a3, (1, 1)))


# speedup vs baseline: 1.0507x; 1.0507x over previous
import functools

import jax
import jax.numpy as jnp
from jax import lax
from jax.experimental import pallas as pl
from jax.experimental.pallas import tpu as pltpu
from jax.experimental.pallas import tpu_sc as plsc

N = 10000
E = 320000
D = 128
G = 64
NC = 2
NS = 16
NW = NC * NS
EPW = E // NW
CH = 100
NCHUNK = EPW // CH
DW = 16
RPT = 624
RTAIL = N - NS * RPT
RB = 2000
NRB = N // RB

def _dot(a, b):
    return jnp.dot(a, b, preferred_element_type=jnp.float32)


@functools.lru_cache(maxsize=None)
def _make_prop():
    mesh = plsc.VectorSubcoreMesh(core_axis_name="c", subcore_axis_name="s")

    @functools.partial(
        pl.kernel,
        out_type=jax.ShapeDtypeStruct((NC * N, D), jnp.float32),
        mesh=mesh,
        scratch_types=[
            pltpu.VMEM_SHARED((N, D), jnp.float32),
            [pltpu.VMEM((CH,), jnp.int32) for _ in range(2)],
            [pltpu.VMEM((CH,), jnp.int32) for _ in range(2)],
            [pltpu.VMEM((CH, D), jnp.float32) for _ in range(2)],
            [pltpu.SemaphoreType.DMA for _ in range(2)],
            [pltpu.SemaphoreType.DMA for _ in range(2)],
        ],
    )
    def prop(u_hbm, src3, dst3, zeros_hbm, out_hbm,
             acc, sidx, didx, rows, gsem, isem):
        cid = lax.axis_index("c")
        sid = lax.axis_index("s")
        wid = sid * NC + cid
        row0 = sid * RPT

        def stage_idx(j, q):
            pltpu.async_copy(src3.at[wid, j], sidx[q], isem[q])
            pltpu.async_copy(dst3.at[wid, j], didx[q], isem[q])

        def wait_idx(j, q):
            pltpu.make_async_copy(src3.at[wid, j], sidx[q], isem[q]).wait()
            pltpu.make_async_copy(dst3.at[wid, j], didx[q], isem[q]).wait()

        stage_idx(0, 0)
        stage_idx(1, 1)
        pltpu.sync_copy(zeros_hbm.at[pl.ds(row0, RPT)],
                        acc.at[pl.ds(row0, RPT)])

        @pl.when(sid == 0)
        def _():
            pltpu.sync_copy(zeros_hbm.at[pl.ds(NS * RPT, RTAIL)],
                            acc.at[pl.ds(NS * RPT, RTAIL)])

        plsc.subcore_barrier()
        wait_idx(0, 0)
        pltpu.async_copy(u_hbm.at[sidx[0]], rows[0], gsem[0])

        def step(k, carry):
            j0 = 2 * k
            j1 = 2 * k + 1

            @pl.when(j1 < NCHUNK)
            def _():
                wait_idx(j1, 1)
                pltpu.async_copy(u_hbm.at[sidx[1]], rows[1], gsem[1])

            pltpu.make_async_copy(u_hbm.at[sidx[0]], rows[0], gsem[0]).wait()
            pltpu.sync_copy(rows[0], acc.at[didx[0]], add=True)

            @pl.when(j0 + 2 < NCHUNK)
            def _():
                stage_idx(j0 + 2, 0)

            @pl.when(j1 < NCHUNK)
            def _():
                pltpu.make_async_copy(u_hbm.at[sidx[1]], rows[1],
                                      gsem[1]).wait()
                pltpu.sync_copy(rows[1], acc.at[didx[1]], add=True)

            @pl.when(j0 + 2 < NCHUNK)
            def _():
                wait_idx(j0 + 2, 0)
                pltpu.async_copy(u_hbm.at[sidx[0]], rows[0], gsem[0])

            @pl.when(j1 + 2 < NCHUNK)
            def _():
                stage_idx(j1 + 2, 1)

            return carry

        lax.fori_loop(0, (NCHUNK + 1) // 2, step, 0)
        plsc.subcore_barrier()
        pltpu.sync_copy(acc.at[pl.ds(row0, RPT)],
                        out_hbm.at[pl.ds(cid * N + row0, RPT)])

        @pl.when(sid == 0)
        def _():
            pltpu.sync_copy(acc.at[pl.ds(NS * RPT, RTAIL)],
                            out_hbm.at[pl.ds(cid * N + NS * RPT, RTAIL)])

    return prop


@functools.lru_cache(maxsize=None)
def _make_degree():
    mesh = plsc.VectorSubcoreMesh(core_axis_name="c", subcore_axis_name="s")

    @functools.partial(
        pl.kernel,
        out_type=jax.ShapeDtypeStruct((NC * N, D), jnp.float32),
        mesh=mesh,
        scratch_types=[
            pltpu.VMEM_SHARED((N, D), jnp.float32),
            [pltpu.VMEM((CH,), jnp.int32) for _ in range(2)],
            pltpu.VMEM((CH, D), jnp.float32),
            [pltpu.SemaphoreType.DMA for _ in range(2)],
        ],
    )
    def degree(src3, ones_hbm, zeros_hbm, out_hbm, acc, didx, ones_v, isem):
        cid = lax.axis_index("c")
        sid = lax.axis_index("s")
        wid = sid * NC + cid
        row0 = sid * RPT

        pltpu.async_copy(src3.at[wid, 0], didx[0], isem[0])
        pltpu.async_copy(src3.at[wid, 1], didx[1], isem[1])
        pltpu.sync_copy(ones_hbm, ones_v)
        pltpu.sync_copy(zeros_hbm.at[pl.ds(row0, RPT)],
                        acc.at[pl.ds(row0, RPT)])

        @pl.when(sid == 0)
        def _():
            pltpu.sync_copy(zeros_hbm.at[pl.ds(NS * RPT, RTAIL)],
                            acc.at[pl.ds(NS * RPT, RTAIL)])

        plsc.subcore_barrier()

        def step(k, carry):
            j0 = 2 * k
            j1 = 2 * k + 1
            pltpu.make_async_copy(src3.at[wid, j0], didx[0], isem[0]).wait()
            pltpu.sync_copy(ones_v, acc.at[didx[0]], add=True)

            @pl.when(j0 + 2 < NCHUNK)
            def _():
                pltpu.async_copy(src3.at[wid, j0 + 2], didx[0], isem[0])

            pltpu.make_async_copy(src3.at[wid, j1], didx[1], isem[1]).wait()
            pltpu.sync_copy(ones_v, acc.at[didx[1]], add=True)

            @pl.when(j1 + 2 < NCHUNK)
            def _():
                pltpu.async_copy(src3.at[wid, j1 + 2], didx[1], isem[1])

            return carry

        lax.fori_loop(0, (NCHUNK + 1) // 2, step, 0)
        plsc.subcore_barrier()
        pltpu.sync_copy(acc.at[pl.ds(row0, RPT)],
                        out_hbm.at[pl.ds(cid * N + row0, RPT)])

        @pl.when(sid == 0)
        def _():
            pltpu.sync_copy(acc.at[pl.ds(NS * RPT, RTAIL)],
                            out_hbm.at[pl.ds(cid * N + NS * RPT, RTAIL)])

    return degree


def _prelude_body(d0, d1, x, dinv_ref, u0_ref):
    deg = d0[...] + d1[...]
    dinv = jnp.where(deg > 0.0, lax.rsqrt(deg), 0.0)
    dinv_ref[...] = dinv
    u0_ref[...] = dinv * x[...]


def _tc_prelude(Dp, x):
    return pl.pallas_call(
        _prelude_body,
        grid=(NRB,),
        in_specs=[
            pl.BlockSpec((RB, D), lambda r: (r, 0)),
            pl.BlockSpec((RB, D), lambda r: (r + NRB, 0)),
            pl.BlockSpec((RB, D), lambda r: (r, 0)),
        ],
        out_specs=[
            pl.BlockSpec((RB, D), lambda r: (r, 0)),
            pl.BlockSpec((RB, D), lambda r: (r, 0)),
        ],
        out_shape=[
            jax.ShapeDtypeStruct((N, D), jnp.float32),
            jax.ShapeDtypeStruct((N, D), jnp.float32),
        ],
    )(Dp, Dp, x)


def _mid_body(p0, p1, dv, h, w0, w1, u1_ref, acc_ref):
    tx1 = -dv[...] * (p0[...] + p1[...])
    u1_ref[...] = dv[...] * tx1
    acc_ref[...] = _dot(h[...], w0[...]) + _dot(tx1, w1[...])


def _tc_mid(P, dinvb, h, W0, W1):
    return pl.pallas_call(
        _mid_body,
        grid=(NRB,),
        in_specs=[
            pl.BlockSpec((RB, D), lambda r: (r, 0)),
            pl.BlockSpec((RB, D), lambda r: (r + NRB, 0)),
            pl.BlockSpec((RB, D), lambda r: (r, 0)),
            pl.BlockSpec((RB, D), lambda r: (r, 0)),
            pl.BlockSpec((D, D), lambda r: (0, 0)),
            pl.BlockSpec((D, D), lambda r: (0, 0)),
        ],
        out_specs=[
            pl.BlockSpec((RB, D), lambda r: (r, 0)),
            pl.BlockSpec((RB, D), lambda r: (r, 0)),
        ],
        out_shape=[
            jax.ShapeDtypeStruct((N, D), jnp.float32),
            jax.ShapeDtypeStruct((N, D), jnp.float32),
        ],
    )(P, P, dinvb, h, W0, W1)


def _end_body(q0, q1, dv, h, acc, w2, b, alpha, hn_ref, un_ref):
    tx2 = -2.0 * dv[...] * (q0[...] + q1[...]) - h[...]
    out = acc[...] + _dot(tx2, w2[...]) + b[...]
    a = alpha[0, 0]
    hn = jnp.where(out >= 0.0, out, a * out)
    hn_ref[...] = hn
    un_ref[...] = dv[...] * hn


def _tc_end(Q, dinvb, h, acc0, W2, b, alpha):
    return pl.pallas_call(
        _end_body,
        grid=(NRB,),
        in_specs=[
            pl.BlockSpec((RB, D), lambda r: (r, 0)),
            pl.BlockSpec((RB, D), lambda r: (r + NRB, 0)),
            pl.BlockSpec((RB, D), lambda r: (r, 0)),
            pl.BlockSpec((RB, D), lambda r: (r, 0)),
            pl.BlockSpec((RB, D), lambda r: (r, 0)),
            pl.BlockSpec((D, D), lambda r: (0, 0)),
            pl.BlockSpec((1, D), lambda r: (0, 0)),
            pl.BlockSpec(memory_space=pltpu.SMEM),
        ],
        out_specs=[
            pl.BlockSpec((RB, D), lambda r: (r, 0)),
            pl.BlockSpec((RB, D), lambda r: (r, 0)),
        ],
        out_shape=[
            jax.ShapeDtypeStruct((N, D), jnp.float32),
            jax.ShapeDtypeStruct((N, D), jnp.float32),
        ],
    )(Q, Q, dinvb, h, acc0, W2, b, alpha)


def _head_body(h, bat, w1, b1, w2, b2, w3, b3, a3, out_ref, g_ref):
    r = pl.program_id(0)

    @pl.when(r == 0)
    def _():
        g_ref[...] = jnp.zeros_like(g_ref)

    bblk = bat[0, 0, :]
    onehot = (lax.broadcasted_iota(jnp.int32, (G, RB), 0)
              == bblk[None, :]).astype(jnp.float32)
    g_ref[...] += _dot(onehot, h[...])

    @pl.when(r == NRB - 1)
    def _():
        g = g_ref[...]
        z = _dot(g, w1[...]) + b1[...]
        a = a3[0, 0]
        z = jnp.where(z >= 0.0, z, a * z)
        z = _dot(z, w2[...]) + b2[...]
        z = 1.0 / (1.0 + jnp.exp(-z))
        z = _dot(z, w3[...]) + b3[...]
        m = jnp.max(z, axis=-1, keepdims=True)
        e = jnp.exp(z - m)
        out_ref[...] = (z - m) - jnp.log(jnp.sum(e, axis=-1, keepdims=True))


def _tc_head(h, batch3, W1, b1, W2, b2, W3, b3, a3):
    return pl.pallas_call(
        _head_body,
        grid=(NRB,),
        in_specs=[
            pl.BlockSpec((RB, D), lambda r: (r, 0)),
            pl.BlockSpec((1, 1, RB), lambda r: (r, 0, 0)),
            pl.BlockSpec((D, D), lambda r: (0, 0)),
            pl.BlockSpec((1, D), lambda r: (0, 0)),
            pl.BlockSpec((D, D // 2), lambda r: (0, 0)),
            pl.BlockSpec((1, D // 2), lambda r: (0, 0)),
            pl.BlockSpec((D // 2, 10), lambda r: (0, 0)),
            pl.BlockSpec((1, 10), lambda r: (0, 0)),
            pl.BlockSpec(memory_space=pltpu.SMEM),
        ],
        out_specs=pl.BlockSpec((G, 10), lambda r: (0, 0)),
        out_shape=jax.ShapeDtypeStruct((G, 10), jnp.float32),
        scratch_shapes=[pltpu.VMEM((G, D), jnp.float32)],
    )(h, batch3, W1, b1, W2, b2, W3, b3, a3)


def kernel(x, edge_index, batch, Wc, bc, a1, a3,
           W_fc1, b_fc1, W_fc2, b_fc2, W_fc3, b_fc3):
    src3 = edge_index[0].astype(jnp.int32).reshape(NW, NCHUNK, CH)
    dst3 = edge_index[1].astype(jnp.int32).reshape(NW, NCHUNK, CH)
    zeros = jnp.zeros((N, D), jnp.float32)
    ones = jnp.ones((CH, D), jnp.float32)
    prop = _make_prop()

    Dp = _make_degree()(src3, ones, zeros)
    dinvb, u = _tc_prelude(Dp, x)

    h = x
    for i in range(4):
        P = prop(u, src3, dst3, zeros)
        alpha = a1 if i == 0 else jnp.float32(0.0)
        u1, acc0 = _tc_mid(P, dinvb, h, Wc[i, 0], Wc[i, 1])
        Q = prop(u1, src3, dst3, zeros)
        h, u = _tc_end(Q, dinvb, h, acc0, Wc[i, 2],
                       bc[i].reshape(1, D), jnp.reshape(alpha, (1, 1)))

    batch3 = batch.reshape(NRB, 1, RB).astype(jnp.int32)
    return _tc_head(h, batch3, W_fc1, b_fc1.reshape(1, D),
                    W_fc2, b_fc2.reshape(1, D // 2),
                    W_fc3, b_fc3.reshape(1, 10), jnp.reshape(a3, (1, 1)))


# speedup vs baseline: 1.0646x; 1.0132x over previous
import functools

import jax
import jax.numpy as jnp
from jax import lax
from jax.experimental import pallas as pl
from jax.experimental.pallas import tpu as pltpu
from jax.experimental.pallas import tpu_sc as plsc

N = 10000
E = 320000
D = 128
G = 64
NC = 2
NS = 16
NW = NC * NS
EPW = E // NW
CH = 125
NCHUNK = EPW // CH
DW = 16
RPT = 624
RTAIL = N - NS * RPT
RB = 2000
NRB = N // RB

def _dot(a, b):
    return jnp.dot(a, b, preferred_element_type=jnp.float32)


@functools.lru_cache(maxsize=None)
def _make_prop():
    mesh = plsc.VectorSubcoreMesh(core_axis_name="c", subcore_axis_name="s")

    @functools.partial(
        pl.kernel,
        out_type=jax.ShapeDtypeStruct((NC * N, D), jnp.float32),
        mesh=mesh,
        scratch_types=[
            pltpu.VMEM_SHARED((N, D), jnp.float32),
            [pltpu.VMEM((CH,), jnp.int32) for _ in range(2)],
            [pltpu.VMEM((CH,), jnp.int32) for _ in range(2)],
            [pltpu.VMEM((CH, D), jnp.float32) for _ in range(2)],
            [pltpu.SemaphoreType.DMA for _ in range(2)],
            [pltpu.SemaphoreType.DMA for _ in range(2)],
        ],
    )
    def prop(u_hbm, src3, dst3, zeros_hbm, out_hbm,
             acc, sidx, didx, rows, gsem, isem):
        cid = lax.axis_index("c")
        sid = lax.axis_index("s")
        wid = sid * NC + cid
        row0 = sid * RPT

        def stage_idx(j, q):
            pltpu.async_copy(src3.at[wid, j], sidx[q], isem[q])
            pltpu.async_copy(dst3.at[wid, j], didx[q], isem[q])

        def wait_idx(j, q):
            pltpu.make_async_copy(src3.at[wid, j], sidx[q], isem[q]).wait()
            pltpu.make_async_copy(dst3.at[wid, j], didx[q], isem[q]).wait()

        stage_idx(0, 0)
        stage_idx(1, 1)
        pltpu.sync_copy(zeros_hbm.at[pl.ds(row0, RPT)],
                        acc.at[pl.ds(row0, RPT)])

        @pl.when(sid == 0)
        def _():
            pltpu.sync_copy(zeros_hbm.at[pl.ds(NS * RPT, RTAIL)],
                            acc.at[pl.ds(NS * RPT, RTAIL)])

        plsc.subcore_barrier()
        wait_idx(0, 0)
        pltpu.async_copy(u_hbm.at[sidx[0]], rows[0], gsem[0])

        def step(k, carry):
            j0 = 2 * k
            j1 = 2 * k + 1

            @pl.when(j1 < NCHUNK)
            def _():
                wait_idx(j1, 1)
                pltpu.async_copy(u_hbm.at[sidx[1]], rows[1], gsem[1])

            pltpu.make_async_copy(u_hbm.at[sidx[0]], rows[0], gsem[0]).wait()
            pltpu.sync_copy(rows[0], acc.at[didx[0]], add=True)

            @pl.when(j0 + 2 < NCHUNK)
            def _():
                stage_idx(j0 + 2, 0)

            @pl.when(j1 < NCHUNK)
            def _():
                pltpu.make_async_copy(u_hbm.at[sidx[1]], rows[1],
                                      gsem[1]).wait()
                pltpu.sync_copy(rows[1], acc.at[didx[1]], add=True)

            @pl.when(j0 + 2 < NCHUNK)
            def _():
                wait_idx(j0 + 2, 0)
                pltpu.async_copy(u_hbm.at[sidx[0]], rows[0], gsem[0])

            @pl.when(j1 + 2 < NCHUNK)
            def _():
                stage_idx(j1 + 2, 1)

            return carry

        lax.fori_loop(0, (NCHUNK + 1) // 2, step, 0)
        plsc.subcore_barrier()
        pltpu.sync_copy(acc.at[pl.ds(row0, RPT)],
                        out_hbm.at[pl.ds(cid * N + row0, RPT)])

        @pl.when(sid == 0)
        def _():
            pltpu.sync_copy(acc.at[pl.ds(NS * RPT, RTAIL)],
                            out_hbm.at[pl.ds(cid * N + NS * RPT, RTAIL)])

    return prop


@functools.lru_cache(maxsize=None)
def _make_degree():
    mesh = plsc.VectorSubcoreMesh(core_axis_name="c", subcore_axis_name="s")

    @functools.partial(
        pl.kernel,
        out_type=jax.ShapeDtypeStruct((NC * N, D), jnp.float32),
        mesh=mesh,
        scratch_types=[
            pltpu.VMEM_SHARED((N, D), jnp.float32),
            [pltpu.VMEM((CH,), jnp.int32) for _ in range(2)],
            pltpu.VMEM((CH, D), jnp.float32),
            [pltpu.SemaphoreType.DMA for _ in range(2)],
        ],
    )
    def degree(src3, ones_hbm, zeros_hbm, out_hbm, acc, didx, ones_v, isem):
        cid = lax.axis_index("c")
        sid = lax.axis_index("s")
        wid = sid * NC + cid
        row0 = sid * RPT

        pltpu.async_copy(src3.at[wid, 0], didx[0], isem[0])
        pltpu.async_copy(src3.at[wid, 1], didx[1], isem[1])
        pltpu.sync_copy(ones_hbm, ones_v)
        pltpu.sync_copy(zeros_hbm.at[pl.ds(row0, RPT)],
                        acc.at[pl.ds(row0, RPT)])

        @pl.when(sid == 0)
        def _():
            pltpu.sync_copy(zeros_hbm.at[pl.ds(NS * RPT, RTAIL)],
                            acc.at[pl.ds(NS * RPT, RTAIL)])

        plsc.subcore_barrier()

        def step(k, carry):
            j0 = 2 * k
            j1 = 2 * k + 1
            pltpu.make_async_copy(src3.at[wid, j0], didx[0], isem[0]).wait()
            pltpu.sync_copy(ones_v, acc.at[didx[0]], add=True)

            @pl.when(j0 + 2 < NCHUNK)
            def _():
                pltpu.async_copy(src3.at[wid, j0 + 2], didx[0], isem[0])

            pltpu.make_async_copy(src3.at[wid, j1], didx[1], isem[1]).wait()
            pltpu.sync_copy(ones_v, acc.at[didx[1]], add=True)

            @pl.when(j1 + 2 < NCHUNK)
            def _():
                pltpu.async_copy(src3.at[wid, j1 + 2], didx[1], isem[1])

            return carry

        lax.fori_loop(0, (NCHUNK + 1) // 2, step, 0)
        plsc.subcore_barrier()
        pltpu.sync_copy(acc.at[pl.ds(row0, RPT)],
                        out_hbm.at[pl.ds(cid * N + row0, RPT)])

        @pl.when(sid == 0)
        def _():
            pltpu.sync_copy(acc.at[pl.ds(NS * RPT, RTAIL)],
                            out_hbm.at[pl.ds(cid * N + NS * RPT, RTAIL)])

    return degree


def _prelude_body(d0, d1, x, dinv_ref, u0_ref):
    deg = d0[...] + d1[...]
    dinv = jnp.where(deg > 0.0, lax.rsqrt(deg), 0.0)
    dinv_ref[...] = dinv
    u0_ref[...] = dinv * x[...]


def _tc_prelude(Dp, x):
    return pl.pallas_call(
        _prelude_body,
        grid=(NRB,),
        in_specs=[
            pl.BlockSpec((RB, D), lambda r: (r, 0)),
            pl.BlockSpec((RB, D), lambda r: (r + NRB, 0)),
            pl.BlockSpec((RB, D), lambda r: (r, 0)),
        ],
        out_specs=[
            pl.BlockSpec((RB, D), lambda r: (r, 0)),
            pl.BlockSpec((RB, D), lambda r: (r, 0)),
        ],
        out_shape=[
            jax.ShapeDtypeStruct((N, D), jnp.float32),
            jax.ShapeDtypeStruct((N, D), jnp.float32),
        ],
    )(Dp, Dp, x)


def _mid_body(p0, p1, dv, h, w0, w1, u1_ref, acc_ref):
    tx1 = -dv[...] * (p0[...] + p1[...])
    u1_ref[...] = dv[...] * tx1
    acc_ref[...] = _dot(h[...], w0[...]) + _dot(tx1, w1[...])


def _tc_mid(P, dinvb, h, W0, W1):
    return pl.pallas_call(
        _mid_body,
        grid=(NRB,),
        in_specs=[
            pl.BlockSpec((RB, D), lambda r: (r, 0)),
            pl.BlockSpec((RB, D), lambda r: (r + NRB, 0)),
            pl.BlockSpec((RB, D), lambda r: (r, 0)),
            pl.BlockSpec((RB, D), lambda r: (r, 0)),
            pl.BlockSpec((D, D), lambda r: (0, 0)),
            pl.BlockSpec((D, D), lambda r: (0, 0)),
        ],
        out_specs=[
            pl.BlockSpec((RB, D), lambda r: (r, 0)),
            pl.BlockSpec((RB, D), lambda r: (r, 0)),
        ],
        out_shape=[
            jax.ShapeDtypeStruct((N, D), jnp.float32),
            jax.ShapeDtypeStruct((N, D), jnp.float32),
        ],
    )(P, P, dinvb, h, W0, W1)


def _end_body(q0, q1, dv, h, acc, w2, b, alpha, hn_ref, un_ref):
    tx2 = -2.0 * dv[...] * (q0[...] + q1[...]) - h[...]
    out = acc[...] + _dot(tx2, w2[...]) + b[...]
    a = alpha[0, 0]
    hn = jnp.where(out >= 0.0, out, a * out)
    hn_ref[...] = hn
    un_ref[...] = dv[...] * hn


def _tc_end(Q, dinvb, h, acc0, W2, b, alpha):
    return pl.pallas_call(
        _end_body,
        grid=(NRB,),
        in_specs=[
            pl.BlockSpec((RB, D), lambda r: (r, 0)),
            pl.BlockSpec((RB, D), lambda r: (r + NRB, 0)),
            pl.BlockSpec((RB, D), lambda r: (r, 0)),
            pl.BlockSpec((RB, D), lambda r: (r, 0)),
            pl.BlockSpec((RB, D), lambda r: (r, 0)),
            pl.BlockSpec((D, D), lambda r: (0, 0)),
            pl.BlockSpec((1, D), lambda r: (0, 0)),
            pl.BlockSpec(memory_space=pltpu.SMEM),
        ],
        out_specs=[
            pl.BlockSpec((RB, D), lambda r: (r, 0)),
            pl.BlockSpec((RB, D), lambda r: (r, 0)),
        ],
        out_shape=[
            jax.ShapeDtypeStruct((N, D), jnp.float32),
            jax.ShapeDtypeStruct((N, D), jnp.float32),
        ],
    )(Q, Q, dinvb, h, acc0, W2, b, alpha)


def _head_body(h, bat, w1, b1, w2, b2, w3, b3, a3, out_ref, g_ref):
    r = pl.program_id(0)

    @pl.when(r == 0)
    def _():
        g_ref[...] = jnp.zeros_like(g_ref)

    bblk = bat[0, 0, :]
    onehot = (lax.broadcasted_iota(jnp.int32, (G, RB), 0)
              == bblk[None, :]).astype(jnp.float32)
    g_ref[...] += _dot(onehot, h[...])

    @pl.when(r == NRB - 1)
    def _():
        g = g_ref[...]
        z = _dot(g, w1[...]) + b1[...]
        a = a3[0, 0]
        z = jnp.where(z >= 0.0, z, a * z)
        z = _dot(z, w2[...]) + b2[...]
        z = 1.0 / (1.0 + jnp.exp(-z))
        z = _dot(z, w3[...]) + b3[...]
        m = jnp.max(z, axis=-1, keepdims=True)
        e = jnp.exp(z - m)
        out_ref[...] = (z - m) - jnp.log(jnp.sum(e, axis=-1, keepdims=True))


def _tc_head(h, batch3, W1, b1, W2, b2, W3, b3, a3):
    return pl.pallas_call(
        _head_body,
        grid=(NRB,),
        in_specs=[
            pl.BlockSpec((RB, D), lambda r: (r, 0)),
            pl.BlockSpec((1, 1, RB), lambda r: (r, 0, 0)),
            pl.BlockSpec((D, D), lambda r: (0, 0)),
            pl.BlockSpec((1, D), lambda r: (0, 0)),
            pl.BlockSpec((D, D // 2), lambda r: (0, 0)),
            pl.BlockSpec((1, D // 2), lambda r: (0, 0)),
            pl.BlockSpec((D // 2, 10), lambda r: (0, 0)),
            pl.BlockSpec((1, 10), lambda r: (0, 0)),
            pl.BlockSpec(memory_space=pltpu.SMEM),
        ],
        out_specs=pl.BlockSpec((G, 10), lambda r: (0, 0)),
        out_shape=jax.ShapeDtypeStruct((G, 10), jnp.float32),
        scratch_shapes=[pltpu.VMEM((G, D), jnp.float32)],
    )(h, batch3, W1, b1, W2, b2, W3, b3, a3)


def kernel(x, edge_index, batch, Wc, bc, a1, a3,
           W_fc1, b_fc1, W_fc2, b_fc2, W_fc3, b_fc3):
    src3 = edge_index[0].astype(jnp.int32).reshape(NW, NCHUNK, CH)
    dst3 = edge_index[1].astype(jnp.int32).reshape(NW, NCHUNK, CH)
    zeros = jnp.zeros((N, D), jnp.float32)
    ones = jnp.ones((CH, D), jnp.float32)
    prop = _make_prop()

    Dp = _make_degree()(src3, ones, zeros)
    dinvb, u = _tc_prelude(Dp, x)

    h = x
    for i in range(4):
        P = prop(u, src3, dst3, zeros)
        alpha = a1 if i == 0 else jnp.float32(0.0)
        u1, acc0 = _tc_mid(P, dinvb, h, Wc[i, 0], Wc[i, 1])
        Q = prop(u1, src3, dst3, zeros)
        h, u = _tc_end(Q, dinvb, h, acc0, Wc[i, 2],
                       bc[i].reshape(1, D), jnp.reshape(alpha, (1, 1)))

    batch3 = batch.reshape(NRB, 1, RB).astype(jnp.int32)
    return _tc_head(h, batch3, W_fc1, b_fc1.reshape(1, D),
                    W_fc2, b_fc2.reshape(1, D // 2),
                    W_fc3, b_fc3.reshape(1, 10), jnp.reshape(a3, (1, 1)))


# speedup vs baseline: 1.0711x; 1.0061x over previous
import functools

import jax
import jax.numpy as jnp
from jax import lax
from jax.experimental import pallas as pl
from jax.experimental.pallas import tpu as pltpu
from jax.experimental.pallas import tpu_sc as plsc

N = 10000
E = 320000
D = 128
G = 64
NC = 2
NS = 16
NW = NC * NS
EPW = E // NW
CH = 125
NCHUNK = EPW // CH
DW = 16
RPT = 624
RTAIL = N - NS * RPT
RB = 2000
NRB = N // RB

def _dot(a, b):
    return jnp.dot(a, b, preferred_element_type=jnp.float32)


@functools.lru_cache(maxsize=None)
def _make_prop():
    mesh = plsc.VectorSubcoreMesh(core_axis_name="c", subcore_axis_name="s")

    @functools.partial(
        pl.kernel,
        out_type=jax.ShapeDtypeStruct((NC * N, D), jnp.float32),
        mesh=mesh,
        scratch_types=[
            pltpu.VMEM_SHARED((N, D), jnp.float32),
            [pltpu.VMEM((CH,), jnp.int32) for _ in range(2)],
            [pltpu.VMEM((CH,), jnp.int32) for _ in range(2)],
            [pltpu.VMEM((CH, D), jnp.float32) for _ in range(2)],
            [pltpu.SemaphoreType.DMA for _ in range(2)],
            [pltpu.SemaphoreType.DMA for _ in range(2)],
        ],
    )
    def prop(u_hbm, src3, dst3, zeros_hbm, out_hbm,
             acc, sidx, didx, rows, gsem, isem):
        cid = lax.axis_index("c")
        sid = lax.axis_index("s")
        wid = sid * NC + cid
        row0 = sid * RPT

        def stage_idx(j, q):
            pltpu.async_copy(src3.at[wid, j], sidx[q], isem[q])
            pltpu.async_copy(dst3.at[wid, j], didx[q], isem[q])

        def wait_idx(j, q):
            pltpu.make_async_copy(src3.at[wid, j], sidx[q], isem[q]).wait()
            pltpu.make_async_copy(dst3.at[wid, j], didx[q], isem[q]).wait()

        stage_idx(0, 0)
        stage_idx(1, 1)
        pltpu.sync_copy(zeros_hbm.at[pl.ds(0, 120)],
                        rows[0].at[pl.ds(0, 120)])
        for z in range(5):
            pltpu.sync_copy(rows[0].at[pl.ds(0, 120)],
                            acc.at[pl.ds(row0 + 120 * z, 120)])
        pltpu.sync_copy(rows[0].at[pl.ds(0, 24)],
                        acc.at[pl.ds(row0 + 600, 24)])

        @pl.when(sid == 0)
        def _():
            pltpu.sync_copy(rows[0].at[pl.ds(0, RTAIL)],
                            acc.at[pl.ds(NS * RPT, RTAIL)])

        plsc.subcore_barrier()
        wait_idx(0, 0)
        pltpu.async_copy(u_hbm.at[sidx[0]], rows[0], gsem[0])

        def step(k, carry):
            j0 = 2 * k
            j1 = 2 * k + 1

            @pl.when(j1 < NCHUNK)
            def _():
                wait_idx(j1, 1)
                pltpu.async_copy(u_hbm.at[sidx[1]], rows[1], gsem[1])

            pltpu.make_async_copy(u_hbm.at[sidx[0]], rows[0], gsem[0]).wait()
            pltpu.sync_copy(rows[0], acc.at[didx[0]], add=True)

            @pl.when(j0 + 2 < NCHUNK)
            def _():
                stage_idx(j0 + 2, 0)

            @pl.when(j1 < NCHUNK)
            def _():
                pltpu.make_async_copy(u_hbm.at[sidx[1]], rows[1],
                                      gsem[1]).wait()
                pltpu.sync_copy(rows[1], acc.at[didx[1]], add=True)

            @pl.when(j0 + 2 < NCHUNK)
            def _():
                wait_idx(j0 + 2, 0)
                pltpu.async_copy(u_hbm.at[sidx[0]], rows[0], gsem[0])

            @pl.when(j1 + 2 < NCHUNK)
            def _():
                stage_idx(j1 + 2, 1)

            return carry

        lax.fori_loop(0, (NCHUNK + 1) // 2, step, 0)
        plsc.subcore_barrier()
        pltpu.sync_copy(acc.at[pl.ds(row0, RPT)],
                        out_hbm.at[pl.ds(cid * N + row0, RPT)])

        @pl.when(sid == 0)
        def _():
            pltpu.sync_copy(acc.at[pl.ds(NS * RPT, RTAIL)],
                            out_hbm.at[pl.ds(cid * N + NS * RPT, RTAIL)])

    return prop


@functools.lru_cache(maxsize=None)
def _make_degree():
    mesh = plsc.VectorSubcoreMesh(core_axis_name="c", subcore_axis_name="s")

    @functools.partial(
        pl.kernel,
        out_type=jax.ShapeDtypeStruct((NC * N, D), jnp.float32),
        mesh=mesh,
        scratch_types=[
            pltpu.VMEM_SHARED((N, D), jnp.float32),
            [pltpu.VMEM((CH,), jnp.int32) for _ in range(2)],
            pltpu.VMEM((CH, D), jnp.float32),
            [pltpu.SemaphoreType.DMA for _ in range(2)],
        ],
    )
    def degree(src3, ones_hbm, zeros_hbm, out_hbm, acc, didx, ones_v, isem):
        cid = lax.axis_index("c")
        sid = lax.axis_index("s")
        wid = sid * NC + cid
        row0 = sid * RPT

        pltpu.async_copy(src3.at[wid, 0], didx[0], isem[0])
        pltpu.async_copy(src3.at[wid, 1], didx[1], isem[1])
        pltpu.sync_copy(ones_hbm, ones_v)
        pltpu.sync_copy(zeros_hbm.at[pl.ds(row0, RPT)],
                        acc.at[pl.ds(row0, RPT)])

        @pl.when(sid == 0)
        def _():
            pltpu.sync_copy(zeros_hbm.at[pl.ds(NS * RPT, RTAIL)],
                            acc.at[pl.ds(NS * RPT, RTAIL)])

        plsc.subcore_barrier()

        def step(k, carry):
            j0 = 2 * k
            j1 = 2 * k + 1
            pltpu.make_async_copy(src3.at[wid, j0], didx[0], isem[0]).wait()
            pltpu.sync_copy(ones_v, acc.at[didx[0]], add=True)

            @pl.when(j0 + 2 < NCHUNK)
            def _():
                pltpu.async_copy(src3.at[wid, j0 + 2], didx[0], isem[0])

            pltpu.make_async_copy(src3.at[wid, j1], didx[1], isem[1]).wait()
            pltpu.sync_copy(ones_v, acc.at[didx[1]], add=True)

            @pl.when(j1 + 2 < NCHUNK)
            def _():
                pltpu.async_copy(src3.at[wid, j1 + 2], didx[1], isem[1])

            return carry

        lax.fori_loop(0, (NCHUNK + 1) // 2, step, 0)
        plsc.subcore_barrier()
        pltpu.sync_copy(acc.at[pl.ds(row0, RPT)],
                        out_hbm.at[pl.ds(cid * N + row0, RPT)])

        @pl.when(sid == 0)
        def _():
            pltpu.sync_copy(acc.at[pl.ds(NS * RPT, RTAIL)],
                            out_hbm.at[pl.ds(cid * N + NS * RPT, RTAIL)])

    return degree


def _prelude_body(d0, d1, x, dinv_ref, u0_ref):
    deg = d0[...] + d1[...]
    dinv = jnp.where(deg > 0.0, lax.rsqrt(deg), 0.0)
    dinv_ref[...] = dinv
    u0_ref[...] = dinv * x[...]


def _tc_prelude(Dp, x):
    return pl.pallas_call(
        _prelude_body,
        grid=(NRB,),
        in_specs=[
            pl.BlockSpec((RB, D), lambda r: (r, 0)),
            pl.BlockSpec((RB, D), lambda r: (r + NRB, 0)),
            pl.BlockSpec((RB, D), lambda r: (r, 0)),
        ],
        out_specs=[
            pl.BlockSpec((RB, D), lambda r: (r, 0)),
            pl.BlockSpec((RB, D), lambda r: (r, 0)),
        ],
        out_shape=[
            jax.ShapeDtypeStruct((N, D), jnp.float32),
            jax.ShapeDtypeStruct((N, D), jnp.float32),
        ],
    )(Dp, Dp, x)


def _mid_body(p0, p1, dv, h, w0, w1, u1_ref, acc_ref):
    tx1 = -dv[...] * (p0[...] + p1[...])
    u1_ref[...] = dv[...] * tx1
    acc_ref[...] = _dot(h[...], w0[...]) + _dot(tx1, w1[...])


def _tc_mid(P, dinvb, h, W0, W1):
    return pl.pallas_call(
        _mid_body,
        grid=(NRB,),
        in_specs=[
            pl.BlockSpec((RB, D), lambda r: (r, 0)),
            pl.BlockSpec((RB, D), lambda r: (r + NRB, 0)),
            pl.BlockSpec((RB, D), lambda r: (r, 0)),
            pl.BlockSpec((RB, D), lambda r: (r, 0)),
            pl.BlockSpec((D, D), lambda r: (0, 0)),
            pl.BlockSpec((D, D), lambda r: (0, 0)),
        ],
        out_specs=[
            pl.BlockSpec((RB, D), lambda r: (r, 0)),
            pl.BlockSpec((RB, D), lambda r: (r, 0)),
        ],
        out_shape=[
            jax.ShapeDtypeStruct((N, D), jnp.float32),
            jax.ShapeDtypeStruct((N, D), jnp.float32),
        ],
    )(P, P, dinvb, h, W0, W1)


def _end_body(q0, q1, dv, h, acc, w2, b, alpha, hn_ref, un_ref):
    tx2 = -2.0 * dv[...] * (q0[...] + q1[...]) - h[...]
    out = acc[...] + _dot(tx2, w2[...]) + b[...]
    a = alpha[0, 0]
    hn = jnp.where(out >= 0.0, out, a * out)
    hn_ref[...] = hn
    un_ref[...] = dv[...] * hn


def _tc_end(Q, dinvb, h, acc0, W2, b, alpha):
    return pl.pallas_call(
        _end_body,
        grid=(NRB,),
        in_specs=[
            pl.BlockSpec((RB, D), lambda r: (r, 0)),
            pl.BlockSpec((RB, D), lambda r: (r + NRB, 0)),
            pl.BlockSpec((RB, D), lambda r: (r, 0)),
            pl.BlockSpec((RB, D), lambda r: (r, 0)),
            pl.BlockSpec((RB, D), lambda r: (r, 0)),
            pl.BlockSpec((D, D), lambda r: (0, 0)),
            pl.BlockSpec((1, D), lambda r: (0, 0)),
            pl.BlockSpec(memory_space=pltpu.SMEM),
        ],
        out_specs=[
            pl.BlockSpec((RB, D), lambda r: (r, 0)),
            pl.BlockSpec((RB, D), lambda r: (r, 0)),
        ],
        out_shape=[
            jax.ShapeDtypeStruct((N, D), jnp.float32),
            jax.ShapeDtypeStruct((N, D), jnp.float32),
        ],
    )(Q, Q, dinvb, h, acc0, W2, b, alpha)


def _head_body(h, bat, w1, b1, w2, b2, w3, b3, a3, out_ref, g_ref):
    r = pl.program_id(0)

    @pl.when(r == 0)
    def _():
        g_ref[...] = jnp.zeros_like(g_ref)

    bblk = bat[0, 0, :]
    onehot = (lax.broadcasted_iota(jnp.int32, (G, RB), 0)
              == bblk[None, :]).astype(jnp.float32)
    g_ref[...] += _dot(onehot, h[...])

    @pl.when(r == NRB - 1)
    def _():
        g = g_ref[...]
        z = _dot(g, w1[...]) + b1[...]
        a = a3[0, 0]
        z = jnp.where(z >= 0.0, z, a * z)
        z = _dot(z, w2[...]) + b2[...]
        z = 1.0 / (1.0 + jnp.exp(-z))
        z = _dot(z, w3[...]) + b3[...]
        m = jnp.max(z, axis=-1, keepdims=True)
        e = jnp.exp(z - m)
        out_ref[...] = (z - m) - jnp.log(jnp.sum(e, axis=-1, keepdims=True))


def _tc_head(h, batch3, W1, b1, W2, b2, W3, b3, a3):
    return pl.pallas_call(
        _head_body,
        grid=(NRB,),
        in_specs=[
            pl.BlockSpec((RB, D), lambda r: (r, 0)),
            pl.BlockSpec((1, 1, RB), lambda r: (r, 0, 0)),
            pl.BlockSpec((D, D), lambda r: (0, 0)),
            pl.BlockSpec((1, D), lambda r: (0, 0)),
            pl.BlockSpec((D, D // 2), lambda r: (0, 0)),
            pl.BlockSpec((1, D // 2), lambda r: (0, 0)),
            pl.BlockSpec((D // 2, 10), lambda r: (0, 0)),
            pl.BlockSpec((1, 10), lambda r: (0, 0)),
            pl.BlockSpec(memory_space=pltpu.SMEM),
        ],
        out_specs=pl.BlockSpec((G, 10), lambda r: (0, 0)),
        out_shape=jax.ShapeDtypeStruct((G, 10), jnp.float32),
        scratch_shapes=[pltpu.VMEM((G, D), jnp.float32)],
    )(h, batch3, W1, b1, W2, b2, W3, b3, a3)


def kernel(x, edge_index, batch, Wc, bc, a1, a3,
           W_fc1, b_fc1, W_fc2, b_fc2, W_fc3, b_fc3):
    src3 = edge_index[0].astype(jnp.int32).reshape(NW, NCHUNK, CH)
    dst3 = edge_index[1].astype(jnp.int32).reshape(NW, NCHUNK, CH)
    zeros = jnp.zeros((N, D), jnp.float32)
    ones = jnp.ones((CH, D), jnp.float32)
    prop = _make_prop()

    Dp = _make_degree()(src3, ones, zeros)
    dinvb, u = _tc_prelude(Dp, x)

    h = x
    for i in range(4):
        P = prop(u, src3, dst3, zeros)
        alpha = a1 if i == 0 else jnp.float32(0.0)
        u1, acc0 = _tc_mid(P, dinvb, h, Wc[i, 0], Wc[i, 1])
        Q = prop(u1, src3, dst3, zeros)
        h, u = _tc_end(Q, dinvb, h, acc0, Wc[i, 2],
                       bc[i].reshape(1, D), jnp.reshape(alpha, (1, 1)))

    batch3 = batch.reshape(NRB, 1, RB).astype(jnp.int32)
    return _tc_head(h, batch3, W_fc1, b_fc1.reshape(1, D),
                    W_fc2, b_fc2.reshape(1, D // 2),
                    W_fc3, b_fc3.reshape(1, 10), jnp.reshape(a3, (1, 1)))
